# Initial kernel scaffold; baseline (speedup 1.0000x reference)
import jax
import jax.numpy as jnp
from jax.experimental import pallas as pl


def kernel(x, table):
    raise NotImplementedError("write your pallas kernel here")


# speedup vs baseline: 1.5009x; 1.5009x over previous
import functools

import jax
import jax.numpy as jnp
from jax import lax
from jax.experimental import pallas as pl
from jax.experimental.pallas import tpu as pltpu
from jax.experimental.pallas import tpu_sc as plsc

EMB_DIM = 768
B_TOTAL = 4 * 2048
NUM_WORKERS = 32
B_PER_W = B_TOTAL // NUM_WORKERS
CHUNK = 64
NBUF = 2
NCHUNKS = B_PER_W // CHUNK

_mesh = plsc.VectorSubcoreMesh(core_axis_name="c", subcore_axis_name="s")


@functools.partial(
    pl.kernel,
    mesh=_mesh,
    out_type=jax.ShapeDtypeStruct((B_TOTAL, EMB_DIM), jnp.float32),
    scratch_types=[
        pltpu.VMEM((B_PER_W,), jnp.int32),
        pltpu.VMEM((NBUF, CHUNK, EMB_DIM), jnp.float32),
        pltpu.SemaphoreType.DMA,
    ],
)
def _emb_lookup(table_hbm, idx_hbm, out_hbm, idx_v, rows_v, sem):
    wid = lax.axis_index("s") * 2 + lax.axis_index("c")
    base = wid * B_PER_W
    pltpu.sync_copy(idx_hbm.at[pl.ds(base, B_PER_W)], idx_v)
    copies = [None] * NBUF
    copies[0] = pltpu.async_copy(
        table_hbm.at[idx_v.at[pl.ds(0, CHUNK)]], rows_v.at[0], sem)
    for ci in range(NCHUNKS):
        buf = ci % NBUF
        nxt = (ci + 1) % NBUF
        if ci + 1 < NCHUNKS:
            copies[nxt] = pltpu.async_copy(
                table_hbm.at[idx_v.at[pl.ds((ci + 1) * CHUNK, CHUNK)]],
                rows_v.at[nxt], sem)
        copies[buf].wait()
        pltpu.sync_copy(rows_v.at[buf],
                        out_hbm.at[pl.ds(base + ci * CHUNK, CHUNK)])


def kernel(x, table):
    flat = x.reshape(-1).astype(jnp.int32)
    out = _emb_lookup(table, flat)
    return out.reshape(x.shape + (EMB_DIM,))
